# baseline (speedup 1.0000x reference)
import functools

import jax
import jax.numpy as jnp
from jax import lax
from jax.experimental import pallas as pl
from jax.experimental.pallas import tpu as pltpu
from jax.experimental.pallas import tpu_sc as plsc

MAXD = 10
EDIM = 32
ROWS = 2 * MAXD + 1

L = 16
NC = 2
NS = 16
NW = NC * NS

NBUF = 2


def _sc_lookup(idxT, tab, J, I):
    JT, IT = J // 8, I // 128
    n_units = JT * IT
    u_per_w = n_units // NW
    n_rounds = u_per_w // NBUF
    mesh = plsc.VectorSubcoreMesh(core_axis_name="c", subcore_axis_name="s")

    @functools.partial(
        pl.kernel,
        mesh=mesh,
        compiler_params=pltpu.CompilerParams(
            needs_layout_passes=False, use_tc_tiling_on_sc=True
        ),
        out_type=jax.ShapeDtypeStruct((J, EDIM, I), jnp.float32),
        scratch_types=[
            pltpu.VMEM((ROWS, EDIM), jnp.float32),
            pltpu.VMEM((ROWS * EDIM,), jnp.float32),
            [pltpu.VMEM((8, 128), jnp.int32) for _ in range(NBUF)],
            [pltpu.VMEM((8, EDIM, 128), jnp.float32) for _ in range(NBUF)],
            [pltpu.SemaphoreType.DMA for _ in range(NBUF)],
            [pltpu.SemaphoreType.DMA for _ in range(NBUF)],
        ],
    )
    def k(idx_hbm, tab_hbm, out_hbm, tab_v, tabf_v, idxs, outs, osems, isems):
        wid = lax.axis_index("s") * NC + lax.axis_index("c")
        u0 = wid * u_per_w
        u_last = u0 + u_per_w - 1

        pltpu.sync_copy(tab_hbm, tab_v)
        for r in range(ROWS):
            for h in range(EDIM // L):
                tabf_v[pl.ds(r * EDIM + h * L, L)] = tab_v[r, pl.ds(h * L, L)]

        def out_slice(u):
            jt = u // IT
            it = u % IT
            return out_hbm.at[pl.ds(jt * 8, 8), :, pl.ds(it * 128, 128)]

        def idx_slice(u):
            jt = u // IT
            it = u % IT
            return idx_hbm.at[pl.ds(jt * 8, 8), pl.ds(it * 128, 128)]

        def start_idx(u, b):
            uc = jnp.minimum(u, u_last)
            pltpu.async_copy(idx_slice(uc), idxs[b], isems[b])

        def run_unit(u, b):
            pltpu.make_async_copy(
                idx_slice(jnp.minimum(u, u_last)), idxs[b], isems[b]
            ).wait()
            idx_v = idxs[b]
            out_b = outs[b]

            @plsc.parallel_loop(0, 64, unroll=4)
            def body(gi):
                jr = gi >> 3
                gl = (gi & 7) * L
                raw = idx_v[jr, pl.ds(gl, L)]
                t32 = (
                    jnp.minimum(jnp.maximum(raw + MAXD, 0), ROWS - 1)
                    * EDIM
                )
                for d in range(EDIM):
                    out_b[jr, d, pl.ds(gl, L)] = plsc.load_gather(
                        tabf_v, [t32 + d]
                    )

            pltpu.async_copy(out_b, out_slice(u), osems[b])
            start_idx(u + NBUF, b)

        def drain(u, b):
            pltpu.make_async_copy(outs[b], out_slice(u), osems[b]).wait()

        for b in range(NBUF):
            start_idx(u0 + b, b)
        for b in range(NBUF):
            run_unit(u0 + b, b)

        def round_body(r, _):
            for b in range(NBUF):
                u = u0 + r * NBUF + b
                drain(u - NBUF, b)
                run_unit(u, b)
            return 0

        lax.fori_loop(1, n_rounds, round_body, 0)

        u_tail = u0 + n_rounds * NBUF
        drain(u_tail - NBUF, 0)
        run_unit(u_tail, 0)
        drain(u_tail - NBUF + 1, 1)
        drain(u_tail, 0)

        pltpu.make_async_copy(idx_slice(u_last), idxs[1], isems[1]).wait()
        pltpu.make_async_copy(idx_slice(u_last), idxs[0], isems[0]).wait()

    return k(idxT, tab)


def kernel(delta_m, delta_embed_weight):
    I, J = delta_m.shape
    idxT = delta_m.T.astype(jnp.int32)
    tab = delta_embed_weight.astype(jnp.float32)
    X = _sc_lookup(idxT, tab, J, I)
    return X.transpose(2, 0, 1)


# speedup vs baseline: 1.2289x; 1.2289x over previous
import functools

import jax
import jax.numpy as jnp
from jax import lax
from jax.experimental import pallas as pl
from jax.experimental.pallas import tpu as pltpu
from jax.experimental.pallas import tpu_sc as plsc

MAXD = 10
EDIM = 32
ROWS = 2 * MAXD + 1

L = 16
NC = 2
NS = 16
NW = NC * NS

NBUF = 2


def _sc_lookup(idxT, tab, J, I):
    JT, IT = J // 8, I // 128
    n_units = JT * IT
    u_per_w = n_units // NW
    n_rounds = u_per_w // NBUF
    mesh = plsc.VectorSubcoreMesh(core_axis_name="c", subcore_axis_name="s")

    @functools.partial(
        pl.kernel,
        mesh=mesh,
        compiler_params=pltpu.CompilerParams(
            needs_layout_passes=False, use_tc_tiling_on_sc=True
        ),
        out_type=jax.ShapeDtypeStruct((J, EDIM, I), jnp.float32),
        scratch_types=[
            pltpu.VMEM((ROWS, EDIM), jnp.float32),
            pltpu.VMEM((ROWS * EDIM,), jnp.float32),
            [pltpu.VMEM((8, 128), jnp.int32) for _ in range(NBUF)],
            [pltpu.VMEM((8, EDIM, 128), jnp.float32) for _ in range(NBUF)],
            [pltpu.SemaphoreType.DMA for _ in range(NBUF)],
            [pltpu.SemaphoreType.DMA for _ in range(NBUF)],
        ],
    )
    def k(idx_hbm, tab_hbm, out_hbm, tab_v, tabf_v, idxs, outs, osems, isems):
        wid = lax.axis_index("s") * NC + lax.axis_index("c")
        u0 = wid * u_per_w
        u_last = u0 + u_per_w - 1

        pltpu.sync_copy(tab_hbm, tab_v)
        for r in range(ROWS):
            for h in range(EDIM // L):
                tabf_v[pl.ds(r * EDIM + h * L, L)] = tab_v[r, pl.ds(h * L, L)]

        def out_slice(u):
            jt = u // IT
            it = u % IT
            return out_hbm.at[pl.ds(jt * 8, 8), :, pl.ds(it * 128, 128)]

        def idx_slice(u):
            jt = u // IT
            it = u % IT
            return idx_hbm.at[pl.ds(jt * 8, 8), pl.ds(it * 128, 128)]

        def start_idx(u, b):
            uc = jnp.minimum(u, u_last)
            pltpu.async_copy(idx_slice(uc), idxs[b], isems[b])

        def run_unit(u, b):
            pltpu.make_async_copy(
                idx_slice(jnp.minimum(u, u_last)), idxs[b], isems[b]
            ).wait()
            idx_v = idxs[b]
            out_b = outs[b]

            @plsc.parallel_loop(0, 64, unroll=2)
            def body(gi):
                jr = gi >> 3
                gl = (gi & 7) * L
                raw = idx_v[jr, pl.ds(gl, L)]
                t32 = (
                    jnp.minimum(jnp.maximum(raw + MAXD, 0), ROWS - 1)
                    * EDIM
                )
                g = plsc.load_gather(tabf_v, [t32])
                for d in range(1, EDIM):
                    g_next = plsc.load_gather(tabf_v, [t32 + d])
                    out_b[jr, d - 1, pl.ds(gl, L)] = g
                    g = g_next
                out_b[jr, EDIM - 1, pl.ds(gl, L)] = g

            pltpu.async_copy(out_b, out_slice(u), osems[b])
            start_idx(u + NBUF, b)

        def drain(u, b):
            pltpu.make_async_copy(outs[b], out_slice(u), osems[b]).wait()

        for b in range(NBUF):
            start_idx(u0 + b, b)
        for b in range(NBUF):
            run_unit(u0 + b, b)

        def round_body(r, _):
            for b in range(NBUF):
                u = u0 + r * NBUF + b
                drain(u - NBUF, b)
                run_unit(u, b)
            return 0

        lax.fori_loop(1, n_rounds, round_body, 0)

        u_tail = u0 + n_rounds * NBUF
        drain(u_tail - NBUF, 0)
        run_unit(u_tail, 0)
        drain(u_tail - NBUF + 1, 1)
        drain(u_tail, 0)

        pltpu.make_async_copy(idx_slice(u_last), idxs[1], isems[1]).wait()
        pltpu.make_async_copy(idx_slice(u_last), idxs[0], isems[0]).wait()

    return k(idxT, tab)


def kernel(delta_m, delta_embed_weight):
    I, J = delta_m.shape
    idxT = delta_m.T.astype(jnp.int32)
    tab = delta_embed_weight.astype(jnp.float32)
    X = _sc_lookup(idxT, tab, J, I)
    return X.transpose(2, 0, 1)


# speedup vs baseline: 1.2344x; 1.0045x over previous
import functools

import jax
import jax.numpy as jnp
from jax import lax
from jax.experimental import pallas as pl
from jax.experimental.pallas import tpu as pltpu
from jax.experimental.pallas import tpu_sc as plsc

MAXD = 10
EDIM = 32
ROWS = 2 * MAXD + 1

L = 16
NC = 2
NS = 16
NW = NC * NS

NBUF = 2


def _sc_lookup(idxT, tab, J, I):
    JT, IT = J // 8, I // 128
    n_units = JT * IT
    u_per_w = n_units // NW
    n_rounds = u_per_w // NBUF
    mesh = plsc.VectorSubcoreMesh(core_axis_name="c", subcore_axis_name="s")

    @functools.partial(
        pl.kernel,
        mesh=mesh,
        compiler_params=pltpu.CompilerParams(
            needs_layout_passes=False, use_tc_tiling_on_sc=True
        ),
        out_type=jax.ShapeDtypeStruct((J, EDIM, I), jnp.float32),
        scratch_types=[
            pltpu.VMEM((ROWS, EDIM), jnp.float32),
            pltpu.VMEM((ROWS * EDIM,), jnp.float32),
            [pltpu.VMEM((8, 128), jnp.int32) for _ in range(NBUF)],
            [pltpu.VMEM((8, EDIM, 128), jnp.float32) for _ in range(NBUF)],
            [pltpu.SemaphoreType.DMA for _ in range(NBUF)],
            [pltpu.SemaphoreType.DMA for _ in range(NBUF)],
        ],
    )
    def k(idx_hbm, tab_hbm, out_hbm, tab_v, tabf_v, idxs, outs, osems, isems):
        wid = lax.axis_index("s") * NC + lax.axis_index("c")
        u0 = wid * u_per_w
        u_last = u0 + u_per_w - 1

        pltpu.sync_copy(tab_hbm, tab_v)
        for r in range(ROWS):
            for h in range(EDIM // L):
                tabf_v[pl.ds(r * EDIM + h * L, L)] = tab_v[r, pl.ds(h * L, L)]

        def out_slice(u):
            jt = u // IT
            it = u % IT
            return out_hbm.at[pl.ds(jt * 8, 8), :, pl.ds(it * 128, 128)]

        def idx_slice(u):
            jt = u // IT
            it = u % IT
            return idx_hbm.at[pl.ds(jt * 8, 8), pl.ds(it * 128, 128)]

        def start_idx(u, b):
            uc = jnp.minimum(u, u_last)
            pltpu.async_copy(idx_slice(uc), idxs[b], isems[b])

        def run_unit(u, b):
            pltpu.make_async_copy(
                idx_slice(jnp.minimum(u, u_last)), idxs[b], isems[b]
            ).wait()
            idx_v = idxs[b]
            out_b = outs[b]

            @plsc.parallel_loop(0, 64, unroll=2)
            def body(gi):
                jr = gi >> 3
                gl = (gi & 7) * L
                raw = idx_v[jr, pl.ds(gl, L)]
                t32 = (
                    jnp.minimum(jnp.maximum(raw + MAXD, 0), ROWS - 1)
                    * EDIM
                )
                g0 = plsc.load_gather(tabf_v, [t32])
                g1 = plsc.load_gather(tabf_v, [t32 + 1])
                for d in range(2, EDIM):
                    g2 = plsc.load_gather(tabf_v, [t32 + d])
                    out_b[jr, d - 2, pl.ds(gl, L)] = g0
                    g0, g1 = g1, g2
                out_b[jr, EDIM - 2, pl.ds(gl, L)] = g0
                out_b[jr, EDIM - 1, pl.ds(gl, L)] = g1

            pltpu.async_copy(out_b, out_slice(u), osems[b])
            start_idx(u + NBUF, b)

        def drain(u, b):
            pltpu.make_async_copy(outs[b], out_slice(u), osems[b]).wait()

        for b in range(NBUF):
            start_idx(u0 + b, b)
        for b in range(NBUF):
            run_unit(u0 + b, b)

        def round_body(r, _):
            for b in range(NBUF):
                u = u0 + r * NBUF + b
                drain(u - NBUF, b)
                run_unit(u, b)
            return 0

        lax.fori_loop(1, n_rounds, round_body, 0)

        u_tail = u0 + n_rounds * NBUF
        drain(u_tail - NBUF, 0)
        run_unit(u_tail, 0)
        drain(u_tail - NBUF + 1, 1)
        drain(u_tail, 0)

        pltpu.make_async_copy(idx_slice(u_last), idxs[1], isems[1]).wait()
        pltpu.make_async_copy(idx_slice(u_last), idxs[0], isems[0]).wait()

    return k(idxT, tab)


def kernel(delta_m, delta_embed_weight):
    I, J = delta_m.shape
    idxT = delta_m.T.astype(jnp.int32)
    tab = delta_embed_weight.astype(jnp.float32)
    X = _sc_lookup(idxT, tab, J, I)
    return X.transpose(2, 0, 1)
